# Initial kernel scaffold; baseline (speedup 1.0000x reference)
import jax
import jax.numpy as jnp
from jax.experimental import pallas as pl


def kernel(x, edge_index, W1l, b1l, W1r, W2l, b2l, W2r):
    raise NotImplementedError("write your pallas kernel here")


# speedup vs baseline: 4.1249x; 4.1249x over previous
import functools

import jax
import jax.numpy as jnp
from jax import lax
from jax.experimental import pallas as pl
from jax.experimental.pallas import tpu as pltpu
from jax.experimental.pallas import tpu_sc as plsc

N_NODES = 10000
N_EDGES = 320000
D = 128

NC = 2
NS = 16
NW = NC * NS
CHUNK = 128
CH_PER_W = 79
E_PAD = NW * CH_PER_W * CHUNK
ACC_ROWS = 10240
ROWS_PER_TILE = ACC_ROWS // NS
D1 = 144
BLK = 400


def _make_agg(d):
    mesh = plsc.VectorSubcoreMesh(core_axis_name="c", subcore_axis_name="s")

    @functools.partial(
        pl.kernel,
        mesh=mesh,
        compiler_params=pltpu.CompilerParams(use_tc_tiling_on_sc=False),
        out_type=jax.ShapeDtypeStruct((NC, ACC_ROWS, d), jnp.float32),
        scratch_types=[
            pltpu.VMEM((CHUNK,), jnp.int32),
            pltpu.VMEM((CHUNK,), jnp.int32),
            pltpu.VMEM((CHUNK, d), jnp.float32),
            pltpu.VMEM_SHARED((ACC_ROWS, d), jnp.float32),
            pltpu.SemaphoreType.DMA,
        ],
    )
    def agg(table_hbm, src_hbm, dst_hbm, zeros_hbm, out_hbm,
            src_v, dst_v, rows_v, acc, sem):
        c = lax.axis_index("c")
        s = lax.axis_index("s")
        wid = c * NS + s
        pltpu.sync_copy(zeros_hbm, acc.at[pl.ds(s * ROWS_PER_TILE, ROWS_PER_TILE)])
        plsc.subcore_barrier()
        base = wid * (CH_PER_W * CHUNK)

        def body(j, carry):
            off = base + j * CHUNK
            pltpu.sync_copy(src_hbm.at[pl.ds(off, CHUNK)], src_v)
            pltpu.sync_copy(dst_hbm.at[pl.ds(off, CHUNK)], dst_v)
            pltpu.async_copy(table_hbm.at[src_v], rows_v, sem).wait()
            pltpu.sync_copy(rows_v, acc.at[dst_v], add=True)
            return carry

        lax.fori_loop(0, CH_PER_W, body, 0)
        plsc.subcore_barrier()
        pltpu.sync_copy(
            acc.at[pl.ds(s * ROWS_PER_TILE, ROWS_PER_TILE)],
            out_hbm.at[c, pl.ds(s * ROWS_PER_TILE, ROWS_PER_TILE)],
        )

    return agg


_agg_l1 = _make_agg(D1)
_agg_l2 = _make_agg(D)


def _dot_t(a, w):
    return lax.dot_general(a, w, (((1,), (1,)), ((), ())),
                           preferred_element_type=jnp.float32)


def _prep1_body(x_ref, w1l_ref, w1r_ref, b1_ref, table_ref, xr_ref):
    xb = x_ref[...]
    t = _dot_t(xb, w1l_ref[...])
    ones = jnp.ones((BLK, 1), jnp.float32)
    pad = jnp.zeros((BLK, D1 - D - 1), jnp.float32)
    table_ref[...] = jnp.concatenate([t, ones, pad], axis=1)
    xr_ref[...] = _dot_t(xb, w1r_ref[...]) + b1_ref[...]


def _prep2_body(p_ref, xr1_ref, w2l_ref, w2r_ref, b2_ref,
                table_ref, xr_ref, inv_ref):
    sm = p_ref[0] + p_ref[1]
    agg = sm[:, 0:D]
    cnt = sm[:, D:D + 1]
    inv = 1.0 / jnp.maximum(cnt, 1.0)
    h = agg * inv + xr1_ref[...]
    table_ref[...] = _dot_t(h, w2l_ref[...])
    xr_ref[...] = _dot_t(h, w2r_ref[...]) + b2_ref[...]
    inv_ref[...] = jnp.broadcast_to(inv, (BLK, D))


def _finish_body(q_ref, inv_ref, xr2_ref, out_ref):
    sm = q_ref[0] + q_ref[1]
    out_ref[...] = sm * inv_ref[...] + xr2_ref[...]


def _prep1(x, w1l, w1r, b1):
    return pl.pallas_call(
        _prep1_body,
        grid=(N_NODES // BLK,),
        in_specs=[
            pl.BlockSpec((BLK, D), lambda i: (i, 0)),
            pl.BlockSpec((D, D), lambda i: (0, 0)),
            pl.BlockSpec((D, D), lambda i: (0, 0)),
            pl.BlockSpec((1, D), lambda i: (0, 0)),
        ],
        out_specs=[
            pl.BlockSpec((BLK, D1), lambda i: (i, 0)),
            pl.BlockSpec((BLK, D), lambda i: (i, 0)),
        ],
        out_shape=[
            jax.ShapeDtypeStruct((N_NODES, D1), jnp.float32),
            jax.ShapeDtypeStruct((N_NODES, D), jnp.float32),
        ],
    )(x, w1l, w1r, b1)


def _prep2(p, xr1, w2l, w2r, b2):
    return pl.pallas_call(
        _prep2_body,
        grid=(N_NODES // BLK,),
        in_specs=[
            pl.BlockSpec((NC, BLK, D1), lambda i: (0, i, 0)),
            pl.BlockSpec((BLK, D), lambda i: (i, 0)),
            pl.BlockSpec((D, D), lambda i: (0, 0)),
            pl.BlockSpec((D, D), lambda i: (0, 0)),
            pl.BlockSpec((1, D), lambda i: (0, 0)),
        ],
        out_specs=[
            pl.BlockSpec((BLK, D), lambda i: (i, 0)),
            pl.BlockSpec((BLK, D), lambda i: (i, 0)),
            pl.BlockSpec((BLK, D), lambda i: (i, 0)),
        ],
        out_shape=[
            jax.ShapeDtypeStruct((N_NODES, D), jnp.float32),
            jax.ShapeDtypeStruct((N_NODES, D), jnp.float32),
            jax.ShapeDtypeStruct((N_NODES, D), jnp.float32),
        ],
    )(p, xr1, w2l, w2r, b2)


def _finish(q, inv, xr2):
    return pl.pallas_call(
        _finish_body,
        grid=(N_NODES // BLK,),
        in_specs=[
            pl.BlockSpec((NC, BLK, D), lambda i: (0, i, 0)),
            pl.BlockSpec((BLK, D), lambda i: (i, 0)),
            pl.BlockSpec((BLK, D), lambda i: (i, 0)),
        ],
        out_specs=pl.BlockSpec((BLK, D), lambda i: (i, 0)),
        out_shape=jax.ShapeDtypeStruct((N_NODES, D), jnp.float32),
    )(q, inv, xr2)


def kernel(x, edge_index, W1l, b1l, W1r, W2l, b2l, W2r):
    ei = edge_index.astype(jnp.int32)
    npad = E_PAD - N_EDGES
    src = jnp.concatenate([ei[0], jnp.zeros((npad,), jnp.int32)])
    dst = jnp.concatenate([ei[1], jnp.full((npad,), N_NODES, jnp.int32)])

    zeros1 = jnp.zeros((ROWS_PER_TILE, D1), jnp.float32)
    zeros2 = jnp.zeros((ROWS_PER_TILE, D), jnp.float32)

    table1, xr1 = _prep1(x, W1l, W1r, b1l.reshape(1, D))
    p = _agg_l1(table1, src, dst, zeros1)
    table2, xr2, inv = _prep2(p, xr1, W2l, W2r, b2l.reshape(1, D))
    q = _agg_l2(table2, src, dst, zeros2)
    return _finish(q, inv, xr2)
